# Initial kernel scaffold; baseline (speedup 1.0000x reference)
import jax
import jax.numpy as jnp
from jax.experimental import pallas as pl


def kernel(x, edge_index, frontier_mask, aggregated_neighbors, target_mask, W1_self, W1_neigh, b1, W2_self, W2_neigh, b2):
    raise NotImplementedError("write your pallas kernel here")


# speedup vs baseline: 3.7380x; 3.7380x over previous
import functools

import jax
import jax.numpy as jnp
from jax import lax
from jax.experimental import pallas as pl
from jax.experimental.pallas import tpu as pltpu
from jax.experimental.pallas import tpu_sc as plsc

_N = 10000
_D = 128
_E = 320000
_NPAD = 10240
_NC = 2
_NS = 16
_NW = _NC * _NS
_K = 128
_EPAD = ((_E + _NW * _K - 1) // (_NW * _K)) * (_NW * _K)
_EPW = _EPAD // _NW
_NCHUNK = _EPW // _K
_RB = 256
_GRID = _NPAD // _RB
_RPT = _NPAD // _NS

_MESH = plsc.VectorSubcoreMesh(core_axis_name="c", subcore_axis_name="s")


@functools.partial(
    pl.kernel,
    out_type=jax.ShapeDtypeStruct((_NC, _NPAD, _D), jnp.float32),
    mesh=_MESH,
    scratch_types=[
        pltpu.VMEM((_K,), jnp.int32),
        pltpu.VMEM((_K,), jnp.int32),
        pltpu.VMEM((_K, _D), jnp.float32),
        pltpu.VMEM_SHARED((_NPAD, _D), jnp.float32),
    ],
)
def _seg_sum(feat_hbm, src_hbm, dst_hbm, zeros_hbm, out_hbm,
             src_v, dst_v, rows_v, acc_sh):
    c = lax.axis_index("c")
    s = lax.axis_index("s")
    wid = s * _NC + c
    r0 = s * _RPT
    pltpu.sync_copy(zeros_hbm.at[pl.ds(r0, _RPT)], acc_sh.at[pl.ds(r0, _RPT)])
    plsc.subcore_barrier()

    def body(i, carry):
        base = wid * _EPW + i * _K
        pltpu.sync_copy(src_hbm.at[pl.ds(base, _K)], src_v)
        pltpu.sync_copy(dst_hbm.at[pl.ds(base, _K)], dst_v)
        pltpu.sync_copy(feat_hbm.at[src_v], rows_v)
        pltpu.sync_copy(rows_v, acc_sh.at[dst_v], add=True)
        return carry

    lax.fori_loop(0, _NCHUNK, body, 0)
    plsc.subcore_barrier()
    pltpu.sync_copy(acc_sh.at[pl.ds(r0, _RPT)], out_hbm.at[c, pl.ds(r0, _RPT)])


@functools.partial(
    pl.kernel,
    out_type=jax.ShapeDtypeStruct((_NC, _NPAD, _D), jnp.float32),
    mesh=_MESH,
    scratch_types=[
        pltpu.VMEM((_K,), jnp.int32),
        pltpu.VMEM((_K, _D), jnp.float32),
        pltpu.VMEM_SHARED((_NPAD, _D), jnp.float32),
    ],
)
def _seg_cnt(dst_hbm, ones_hbm, zeros_hbm, out_hbm, dst_v, ones_v, acc_sh):
    c = lax.axis_index("c")
    s = lax.axis_index("s")
    wid = s * _NC + c
    r0 = s * _RPT
    pltpu.sync_copy(zeros_hbm.at[pl.ds(r0, _RPT)], acc_sh.at[pl.ds(r0, _RPT)])
    pltpu.sync_copy(ones_hbm, ones_v)
    plsc.subcore_barrier()

    def body(i, carry):
        base = wid * _EPW + i * _K
        pltpu.sync_copy(dst_hbm.at[pl.ds(base, _K)], dst_v)
        pltpu.sync_copy(ones_v, acc_sh.at[dst_v], add=True)
        return carry

    lax.fori_loop(0, _NCHUNK, body, 0)
    plsc.subcore_barrier()
    pltpu.sync_copy(acc_sh.at[pl.ds(r0, _RPT)], out_hbm.at[c, pl.ds(r0, _RPT)])


def _any_body(m_ref, o_ref):
    o_ref[...] = jnp.max(m_ref[...])[None, None]


def _prep_body(x_ref, fm_ref, o_ref):
    o_ref[...] = jnp.where(fm_ref[...] > 0.0, 0.0, x_ref[...])


def _layer1_body(xz_ref, p_ref, c_ref, agg_ref, tm_ref, use_ref,
                 ws_ref, wn_ref, b_ref, h_ref, cnt_ref):
    ssum = p_ref[0] + p_ref[1]
    cnt = jnp.max(c_ref[0] + c_ref[1], axis=1, keepdims=True)
    cntc = jnp.maximum(cnt, 1.0)
    mean = ssum / cntc
    h = jnp.dot(xz_ref[...], ws_ref[...], preferred_element_type=jnp.float32)
    h += jnp.dot(mean, wn_ref[...], preferred_element_type=jnp.float32)
    h += b_ref[...]
    pre = jnp.dot(agg_ref[...], wn_ref[...], preferred_element_type=jnp.float32)
    pre += b_ref[...]
    cond = jnp.logical_and(tm_ref[...] > 0.0, use_ref[0, 0] > 0.0)
    h = jnp.where(cond, pre, h)
    h_ref[...] = jnp.maximum(h, 0.0)
    cnt_ref[...] = cntc


def _layer2_body(h_ref, p_ref, cnt_ref, ws_ref, wn_ref, b_ref, o_ref):
    mean = (p_ref[0] + p_ref[1]) / cnt_ref[...]
    o = jnp.dot(h_ref[...], ws_ref[...], preferred_element_type=jnp.float32)
    o += jnp.dot(mean, wn_ref[...], preferred_element_type=jnp.float32)
    o_ref[...] = o + b_ref[...]


def kernel(x, edge_index, frontier_mask, aggregated_neighbors, target_mask,
           W1_self, W1_neigh, b1, W2_self, W2_neigh, b2):
    f32 = jnp.float32
    npd = _NPAD - _N
    x_p = jnp.pad(x, ((0, npd), (0, 0)))
    agg_p = jnp.pad(aggregated_neighbors, ((0, npd), (0, 0)))
    fm = jnp.pad(frontier_mask.astype(f32), (0, npd))
    tm = jnp.pad(target_mask.astype(f32), (0, npd))
    fm_col = fm.reshape(_NPAD, 1)
    tm_col = tm.reshape(_NPAD, 1)
    fm2d = fm.reshape(_NPAD // 128, 128)
    src = jnp.pad(edge_index[0], (0, _EPAD - _E))
    dst = jnp.pad(edge_index[1], (0, _EPAD - _E), constant_values=_N)
    b1r = b1.reshape(1, _D)
    b2r = b2.reshape(1, _D)
    zeros_d = jnp.zeros((_NPAD, _D), f32)
    ones_k = jnp.ones((_K, _D), f32)

    use = pl.pallas_call(
        _any_body,
        out_shape=jax.ShapeDtypeStruct((1, 1), f32),
    )(fm2d)

    xz = pl.pallas_call(
        _prep_body,
        grid=(_GRID,),
        in_specs=[pl.BlockSpec((_RB, _D), lambda i: (i, 0)),
                  pl.BlockSpec((_RB, 1), lambda i: (i, 0))],
        out_specs=pl.BlockSpec((_RB, _D), lambda i: (i, 0)),
        out_shape=jax.ShapeDtypeStruct((_NPAD, _D), f32),
    )(x_p, fm_col)

    cnt_part = _seg_cnt(dst, ones_k, zeros_d)
    part1 = _seg_sum(xz, src, dst, zeros_d)

    h1, cnt = pl.pallas_call(
        _layer1_body,
        grid=(_GRID,),
        in_specs=[
            pl.BlockSpec((_RB, _D), lambda i: (i, 0)),
            pl.BlockSpec((2, _RB, _D), lambda i: (0, i, 0)),
            pl.BlockSpec((2, _RB, _D), lambda i: (0, i, 0)),
            pl.BlockSpec((_RB, _D), lambda i: (i, 0)),
            pl.BlockSpec((_RB, 1), lambda i: (i, 0)),
            pl.BlockSpec((1, 1), lambda i: (0, 0)),
            pl.BlockSpec((_D, _D), lambda i: (0, 0)),
            pl.BlockSpec((_D, _D), lambda i: (0, 0)),
            pl.BlockSpec((1, _D), lambda i: (0, 0)),
        ],
        out_specs=[pl.BlockSpec((_RB, _D), lambda i: (i, 0)),
                   pl.BlockSpec((_RB, 1), lambda i: (i, 0))],
        out_shape=[jax.ShapeDtypeStruct((_NPAD, _D), f32),
                   jax.ShapeDtypeStruct((_NPAD, 1), f32)],
    )(xz, part1, cnt_part, agg_p, tm_col, use, W1_self, W1_neigh, b1r)

    part2 = _seg_sum(h1, src, dst, zeros_d)

    out = pl.pallas_call(
        _layer2_body,
        grid=(_GRID,),
        in_specs=[
            pl.BlockSpec((_RB, _D), lambda i: (i, 0)),
            pl.BlockSpec((2, _RB, _D), lambda i: (0, i, 0)),
            pl.BlockSpec((_RB, 1), lambda i: (i, 0)),
            pl.BlockSpec((_D, _D), lambda i: (0, 0)),
            pl.BlockSpec((_D, _D), lambda i: (0, 0)),
            pl.BlockSpec((1, _D), lambda i: (0, 0)),
        ],
        out_specs=pl.BlockSpec((_RB, _D), lambda i: (i, 0)),
        out_shape=jax.ShapeDtypeStruct((_NPAD, _D), f32),
    )(h1, part2, cnt, W2_self, W2_neigh, b2r)

    return out[:_N]
